# baseline (speedup 1.0000x reference)
import functools

import jax
import jax.numpy as jnp
from jax.experimental import pallas as pl
from jax.experimental.pallas import tpu as pltpu

E = 8
K = 2
D_IN = 1024
D_H = 256
D_OUT = 1024
T = 2048

BT = 256
EPAD = 128
NEG = -1e30


def _moe_kernel(x_ref, wg_ref, bg_ref, w1_ref, b1_ref, w2_ref, b2_ref,
                final_ref, masked_ref, gates_ref, idx_ref,
                a1_s, a2_s, w0_s, w1s_s):
    e = pl.program_id(1)

    @pl.when(e == 0)
    def _gating():
        logits = jnp.dot(x_ref[...], wg_ref[...],
                         preferred_element_type=jnp.float32) + bg_ref[...]
        m = jnp.max(logits, axis=1, keepdims=True)
        p = jnp.exp(logits - m)
        g = p / jnp.sum(p, axis=1, keepdims=True)
        gates_ref[...] = g
        lane = jax.lax.broadcasted_iota(jnp.int32, g.shape, 1)
        m1 = jnp.max(g, axis=1, keepdims=True)
        a1 = jnp.min(jnp.where(g == m1, lane, EPAD), axis=1, keepdims=True)
        g2 = jnp.where(lane == a1, -1.0, g)
        m2 = jnp.max(g2, axis=1, keepdims=True)
        a2 = jnp.min(jnp.where(g2 == m2, lane, EPAD), axis=1, keepdims=True)
        s = m1 + m2
        a1_s[...] = a1
        a2_s[...] = a2
        w0_s[...] = m1 / s
        w1s_s[...] = m2 / s
        idx_ref[...] = jnp.where(lane == 0, a1,
                                 jnp.where(lane == 1, a2, 0))

    h = jnp.maximum(
        jnp.dot(x_ref[...], w1_ref[0], preferred_element_type=jnp.float32)
        + b1_ref[0], 0.0)
    out = jnp.dot(h, w2_ref[0], preferred_element_type=jnp.float32) + b2_ref[0]

    sel1 = a1_s[...] == e
    sel2 = a2_s[...] == e
    colm = (sel1 | sel2).astype(jnp.float32)
    colw = jnp.where(sel1, w0_s[...], 0.0) + jnp.where(sel2, w1s_s[...], 0.0)

    masked_ref[0] = out * colm

    @pl.when(e == 0)
    def _init():
        final_ref[...] = colw * out

    @pl.when(e > 0)
    def _acc():
        final_ref[...] += colw * out


@jax.jit
def kernel(x, Wg, bg, W1, b1, W2, b2):
    wg_pad = jnp.zeros((D_IN, EPAD), jnp.float32).at[:, :E].set(Wg)
    bg_pad = jnp.full((1, EPAD), NEG, jnp.float32).at[0, :E].set(bg)
    b1r = b1[:, None, :]
    b2r = b2[:, None, :]

    grid = (T // BT, E)
    out_shapes = (
        jax.ShapeDtypeStruct((T, D_OUT), jnp.float32),
        jax.ShapeDtypeStruct((E, T, D_OUT), jnp.float32),
        jax.ShapeDtypeStruct((T, EPAD), jnp.float32),
        jax.ShapeDtypeStruct((T, EPAD), jnp.int32),
    )
    final, masked, gates_pad, idx_pad = pl.pallas_call(
        _moe_kernel,
        grid=grid,
        in_specs=[
            pl.BlockSpec((BT, D_IN), lambda t, e: (t, 0)),
            pl.BlockSpec((D_IN, EPAD), lambda t, e: (0, 0)),
            pl.BlockSpec((1, EPAD), lambda t, e: (0, 0)),
            pl.BlockSpec((1, D_IN, D_H), lambda t, e: (e, 0, 0)),
            pl.BlockSpec((1, 1, D_H), lambda t, e: (e, 0, 0)),
            pl.BlockSpec((1, D_H, D_OUT), lambda t, e: (e, 0, 0)),
            pl.BlockSpec((1, 1, D_OUT), lambda t, e: (e, 0, 0)),
        ],
        out_specs=(
            pl.BlockSpec((BT, D_OUT), lambda t, e: (t, 0)),
            pl.BlockSpec((1, BT, D_OUT), lambda t, e: (e, t, 0)),
            pl.BlockSpec((BT, EPAD), lambda t, e: (t, 0)),
            pl.BlockSpec((BT, EPAD), lambda t, e: (t, 0)),
        ),
        out_shape=out_shapes,
        scratch_shapes=[
            pltpu.VMEM((BT, 1), jnp.int32),
            pltpu.VMEM((BT, 1), jnp.int32),
            pltpu.VMEM((BT, 1), jnp.float32),
            pltpu.VMEM((BT, 1), jnp.float32),
        ],
        compiler_params=pltpu.CompilerParams(
            dimension_semantics=("arbitrary", "arbitrary"),
        ),
    )(x, wg_pad, bg_pad, W1, b1r, W2, b2r)

    return (final, masked, gates_pad[:, :E], idx_pad[:, :K])


# speedup vs baseline: 1.8063x; 1.8063x over previous
import jax
import jax.numpy as jnp
from jax.experimental import pallas as pl
from jax.experimental.pallas import tpu as pltpu

E = 8
K = 2
D_IN = 1024
D_H = 256
D_OUT = 1024
T = 2048

EPAD = 128
NEG = -1e30


def _moe_kernel(x_ref, wg_ref, bg_ref, w1_ref, b1_ref, w2_ref, b2_ref,
                final_ref, masked_ref, gates_ref, idx_ref,
                a1_s, a2_s, w0_s, w1s_s):
    e = pl.program_id(0)

    @pl.when(e == 0)
    def _gating():
        logits = jnp.dot(x_ref[...], wg_ref[...],
                         preferred_element_type=jnp.float32) + bg_ref[...]
        m = jnp.max(logits, axis=1, keepdims=True)
        p = jnp.exp(logits - m)
        g = p / jnp.sum(p, axis=1, keepdims=True)
        gates_ref[...] = g
        lane = jax.lax.broadcasted_iota(jnp.int32, g.shape, 1)
        m1 = jnp.max(g, axis=1, keepdims=True)
        a1 = jnp.min(jnp.where(g == m1, lane, EPAD), axis=1, keepdims=True)
        g2 = jnp.where(lane == a1, -1.0, g)
        m2 = jnp.max(g2, axis=1, keepdims=True)
        a2 = jnp.min(jnp.where(g2 == m2, lane, EPAD), axis=1, keepdims=True)
        s = m1 + m2
        a1_s[...] = a1
        a2_s[...] = a2
        w0_s[...] = m1 / s
        w1s_s[...] = m2 / s
        idx_ref[...] = jnp.where(lane == 0, a1,
                                 jnp.where(lane == 1, a2, 0))

    h = jnp.maximum(
        jnp.dot(x_ref[...], w1_ref[0], preferred_element_type=jnp.float32)
        + b1_ref[0], 0.0)
    out = jnp.dot(h, w2_ref[0], preferred_element_type=jnp.float32) + b2_ref[0]

    sel1 = a1_s[...] == e
    sel2 = a2_s[...] == e
    colm = (sel1 | sel2).astype(jnp.float32)
    colw = jnp.where(sel1, w0_s[...], 0.0) + jnp.where(sel2, w1s_s[...], 0.0)

    masked_ref[0] = out * colm

    @pl.when(e == 0)
    def _init():
        final_ref[...] = colw * out

    @pl.when(e > 0)
    def _acc():
        final_ref[...] += colw * out


@jax.jit
def kernel(x, Wg, bg, W1, b1, W2, b2):
    wg_pad = jnp.zeros((D_IN, EPAD), jnp.float32).at[:, :E].set(Wg)
    bg_pad = jnp.full((1, EPAD), NEG, jnp.float32).at[0, :E].set(bg)
    b1r = b1[:, None, :]
    b2r = b2[:, None, :]

    out_shapes = (
        jax.ShapeDtypeStruct((T, D_OUT), jnp.float32),
        jax.ShapeDtypeStruct((E, T, D_OUT), jnp.float32),
        jax.ShapeDtypeStruct((T, EPAD), jnp.float32),
        jax.ShapeDtypeStruct((T, EPAD), jnp.int32),
    )
    final, masked, gates_pad, idx_pad = pl.pallas_call(
        _moe_kernel,
        grid=(E,),
        in_specs=[
            pl.BlockSpec((T, D_IN), lambda e: (0, 0)),
            pl.BlockSpec((D_IN, EPAD), lambda e: (0, 0)),
            pl.BlockSpec((1, EPAD), lambda e: (0, 0)),
            pl.BlockSpec((1, D_IN, D_H), lambda e: (e, 0, 0)),
            pl.BlockSpec((1, 1, D_H), lambda e: (e, 0, 0)),
            pl.BlockSpec((1, D_H, D_OUT), lambda e: (e, 0, 0)),
            pl.BlockSpec((1, 1, D_OUT), lambda e: (e, 0, 0)),
        ],
        out_specs=(
            pl.BlockSpec((T, D_OUT), lambda e: (0, 0)),
            pl.BlockSpec((1, T, D_OUT), lambda e: (e, 0, 0)),
            pl.BlockSpec((T, EPAD), lambda e: (0, 0)),
            pl.BlockSpec((T, EPAD), lambda e: (0, 0)),
        ),
        out_shape=out_shapes,
        scratch_shapes=[
            pltpu.VMEM((T, 1), jnp.int32),
            pltpu.VMEM((T, 1), jnp.int32),
            pltpu.VMEM((T, 1), jnp.float32),
            pltpu.VMEM((T, 1), jnp.float32),
        ],
        compiler_params=pltpu.CompilerParams(
            dimension_semantics=("arbitrary",),
        ),
    )(x, wg_pad, bg_pad, W1, b1r, W2, b2r)

    return (final, masked, gates_pad[:, :E], idx_pad[:, :K])
